# initial kernel scaffold (unmeasured)
import jax
import jax.numpy as jnp
from jax import lax
from jax.experimental import pallas as pl
from jax.experimental.pallas import tpu as pltpu

N_DEV = 32

_sem_signal = getattr(pl, "semaphore_signal", None) or pltpu.semaphore_signal
_sem_wait = getattr(pl, "semaphore_wait", None) or pltpu.semaphore_wait
_DevIdType = getattr(pl, "DeviceIdType", None) or pltpu.DeviceIdType
_MESH = _DevIdType.MESH


def kernel(x, w_mat, scale_x, scale_w):
    m, k_loc = x.shape
    _, n = w_mat.shape
    chunk = m // N_DEV

    def body(x_ref, w_ref, sx_ref, sw_ref, out_ref,
             comm_ref, pbuf_ref, wb_ref, send_sems, recv_sems,
             credit_sem, store_sem):
        my = lax.axis_index("i")
        left = lax.rem(my - 1 + N_DEV, N_DEV)
        right = lax.rem(my + 1, N_DEV)
        scale = sx_ref[0] * sw_ref[0]

        barrier_sem = pltpu.get_barrier_semaphore()
        _sem_signal(barrier_sem, inc=1, device_id=(left,), device_id_type=_MESH)
        _sem_signal(barrier_sem, inc=1, device_id=(right,), device_id_type=_MESH)
        _sem_wait(barrier_sem, 2)

        wb_ref[...] = w_ref[...].astype(jnp.bfloat16)

        def partial_chunk(c):
            xc = x_ref[pl.ds(c * chunk, chunk), :].astype(jnp.bfloat16)
            acc = jnp.dot(xc, wb_ref[...], preferred_element_type=jnp.float32)
            return acc * scale

        def ring_step(g, do_accum, c_accum, c_store):
            send_slot = lax.rem(g, 2)
            recv_slot = lax.rem(g + 1, 2)

            @pl.when(g >= 1)
            def _():
                _sem_wait(credit_sem, 1)

            rdma = pltpu.make_async_remote_copy(
                src_ref=comm_ref.at[send_slot],
                dst_ref=comm_ref.at[recv_slot],
                send_sem=send_sems.at[send_slot],
                recv_sem=recv_sems.at[recv_slot],
                device_id=(right,),
                device_id_type=_MESH,
            )
            rdma.start()
            if do_accum:
                pbuf_ref[...] = partial_chunk(c_accum)
            rdma.wait()
            if do_accum:
                comm_ref[recv_slot] = comm_ref[recv_slot] + pbuf_ref[...]
            else:
                store = pltpu.make_async_copy(
                    comm_ref.at[recv_slot],
                    out_ref.at[pl.ds(c_store * chunk, chunk)],
                    store_sem,
                )
                store.start()
                store.wait()
            _sem_signal(credit_sem, inc=1, device_id=(left,),
                        device_id_type=_MESH)

        comm_ref[0] = partial_chunk(my)

        def rs_step(s, carry):
            c_recv = lax.rem(my - s - 1 + 2 * N_DEV, N_DEV)
            ring_step(s, True, c_recv, 0)
            return carry

        lax.fori_loop(0, N_DEV - 1, rs_step, 0)

        red_c = lax.rem(my + 1, N_DEV)
        store = pltpu.make_async_copy(
            comm_ref.at[1], out_ref.at[pl.ds(red_c * chunk, chunk)], store_sem
        )
        store.start()
        store.wait()

        def ag_step(t, carry):
            c_recv = lax.rem(my - t + 2 * N_DEV, N_DEV)
            ring_step(t + N_DEV - 1, False, 0, c_recv)
            return carry

        lax.fori_loop(0, N_DEV - 1, ag_step, 0)

        _sem_wait(credit_sem, 1)

    return pl.pallas_call(
        body,
        out_shape=jax.ShapeDtypeStruct((m, n), jnp.float32),
        in_specs=[
            pl.BlockSpec(memory_space=pltpu.VMEM),
            pl.BlockSpec(memory_space=pltpu.VMEM),
            pl.BlockSpec(memory_space=pltpu.SMEM),
            pl.BlockSpec(memory_space=pltpu.SMEM),
        ],
        out_specs=pl.BlockSpec(memory_space=pltpu.ANY),
        scratch_shapes=[
            pltpu.VMEM((2, chunk, n), jnp.float32),
            pltpu.VMEM((chunk, n), jnp.float32),
            pltpu.VMEM((k_loc, n), jnp.bfloat16),
            pltpu.SemaphoreType.DMA((2,)),
            pltpu.SemaphoreType.DMA((2,)),
            pltpu.SemaphoreType.REGULAR,
            pltpu.SemaphoreType.DMA,
        ],
        compiler_params=pltpu.CompilerParams(collective_id=0),
    )(x, w_mat, scale_x, scale_w)


# baseline (device time: 3392910 ns/iter reference)
import jax
import jax.numpy as jnp
from jax import lax
from jax.experimental import pallas as pl
from jax.experimental.pallas import tpu as pltpu

N_DEV = 32

_sem_signal = getattr(pl, "semaphore_signal", None) or pltpu.semaphore_signal
_sem_wait = getattr(pl, "semaphore_wait", None) or pltpu.semaphore_wait
_DevIdType = getattr(pl, "DeviceIdType", None) or pltpu.DeviceIdType
_MESH = _DevIdType.MESH


def kernel(x, w_mat, scale_x, scale_w):
    m, k_loc = x.shape
    _, n = w_mat.shape
    chunk = m // N_DEV

    def body(x_ref, w_ref, sx_ref, sw_ref, out_ref,
             comm_ref, pbuf_ref, wb_ref, send_sems, recv_sems,
             credit_sem, store_sem):
        my = lax.axis_index("i")
        left = lax.rem(my - 1 + N_DEV, N_DEV)
        right = lax.rem(my + 1, N_DEV)
        scale = sx_ref[0] * sw_ref[0]

        barrier_sem = pltpu.get_barrier_semaphore()
        _sem_signal(barrier_sem, inc=1, device_id=(left,), device_id_type=_MESH)
        _sem_signal(barrier_sem, inc=1, device_id=(right,), device_id_type=_MESH)
        _sem_wait(barrier_sem, 2)

        wb_ref[...] = w_ref[...].astype(jnp.bfloat16)

        def partial_chunk(c):
            xc = x_ref[pl.ds(c * chunk, chunk), :].astype(jnp.bfloat16)
            acc = jnp.dot(xc, wb_ref[...], preferred_element_type=jnp.float32)
            return acc * scale

        def ring_step(g, do_accum, c_accum, c_store):
            send_slot = lax.rem(g, 2)
            recv_slot = lax.rem(g + 1, 2)

            @pl.when(g >= 1)
            def _():
                _sem_wait(credit_sem, 1)

            rdma = pltpu.make_async_remote_copy(
                src_ref=comm_ref.at[send_slot],
                dst_ref=comm_ref.at[recv_slot],
                send_sem=send_sems.at[send_slot],
                recv_sem=recv_sems.at[recv_slot],
                device_id=(right,),
                device_id_type=_MESH,
            )
            rdma.start()
            if do_accum:
                pbuf_ref[...] = partial_chunk(c_accum)
            rdma.wait()
            if do_accum:
                comm_ref[recv_slot] = comm_ref[recv_slot] + pbuf_ref[...]
            else:
                store = pltpu.make_async_copy(
                    comm_ref.at[recv_slot],
                    out_ref.at[pl.ds(c_store * chunk, chunk)],
                    store_sem,
                )
                store.start()
                store.wait()
            _sem_signal(credit_sem, inc=1, device_id=(left,),
                        device_id_type=_MESH)

        comm_ref[0] = partial_chunk(my)

        def rs_step(s, carry):
            c_recv = lax.rem(my - s - 1 + 2 * N_DEV, N_DEV)
            ring_step(s, True, c_recv, 0)
            return carry

        lax.fori_loop(0, N_DEV - 1, rs_step, 0)

        red_c = lax.rem(my + 1, N_DEV)
        store = pltpu.make_async_copy(
            comm_ref.at[1], out_ref.at[pl.ds(red_c * chunk, chunk)], store_sem
        )
        store.start()
        store.wait()

        def ag_step(t, carry):
            c_recv = lax.rem(my - t + 2 * N_DEV, N_DEV)
            ring_step(t + N_DEV - 1, False, 0, c_recv)
            return carry

        lax.fori_loop(0, N_DEV - 1, ag_step, 0)

        _sem_wait(credit_sem, 1)

    return pl.pallas_call(
        body,
        out_shape=jax.ShapeDtypeStruct((m, n), jnp.float32),
        in_specs=[
            pl.BlockSpec(memory_space=pltpu.VMEM),
            pl.BlockSpec(memory_space=pltpu.VMEM),
            pl.BlockSpec(memory_space=pltpu.SMEM),
            pl.BlockSpec(memory_space=pltpu.SMEM),
        ],
        out_specs=pl.BlockSpec(memory_space=pl.ANY),
        scratch_shapes=[
            pltpu.VMEM((2, chunk, n), jnp.float32),
            pltpu.VMEM((chunk, n), jnp.float32),
            pltpu.VMEM((k_loc, n), jnp.bfloat16),
            pltpu.SemaphoreType.DMA((2,)),
            pltpu.SemaphoreType.DMA((2,)),
            pltpu.SemaphoreType.REGULAR,
            pltpu.SemaphoreType.DMA,
        ],
        compiler_params=pltpu.CompilerParams(collective_id=0),
    )(x, w_mat, scale_x, scale_w)


# device time: 1720991 ns/iter; 1.9715x vs baseline; 1.9715x over previous
import jax
import jax.numpy as jnp
import numpy as np
from jax import lax
from jax.experimental import pallas as pl
from jax.experimental.pallas import tpu as pltpu

N_DEV = 32

_sem_signal = getattr(pl, "semaphore_signal", None) or pltpu.semaphore_signal
_sem_wait = getattr(pl, "semaphore_wait", None) or pltpu.semaphore_wait
_DevIdType = getattr(pl, "DeviceIdType", None) or pltpu.DeviceIdType
_MESH = _DevIdType.MESH


def _ring_tables():
    path_yz = []
    for z in range(4):
        ys = range(4) if z % 2 == 0 else range(3, -1, -1)
        path_yz.extend((y, z) for y in ys)
    cycle = [(0, y, z) for (y, z) in path_yz]
    cycle += [(1, y, z) for (y, z) in reversed(path_yz)]
    off = {(0, 0): 0, (1, 0): 1, (1, 1): 2, (0, 1): 3,
           (0, 2): 4, (1, 2): 5, (1, 3): 6, (0, 3): 7}
    perm = np.array([z * 8 + off[(x, y)] for (x, y, z) in cycle],
                    dtype=np.int32)
    inv = np.zeros_like(perm)
    inv[perm] = np.arange(N_DEV, dtype=np.int32)
    return perm, inv


_PERM, _INV = _ring_tables()


def kernel(x, w_mat, scale_x, scale_w):
    m, k_loc = x.shape
    _, n = w_mat.shape
    chunk = m // N_DEV
    nh = n // 2

    my_log = lax.axis_index("i")
    rpos = jnp.take(jnp.asarray(_INV), my_log)
    right_log = jnp.take(jnp.asarray(_PERM), (rpos + 1) % N_DEV)
    left_log = jnp.take(jnp.asarray(_PERM), (rpos + N_DEV - 1) % N_DEV)
    ids = jnp.stack([rpos, left_log, right_log]).astype(jnp.int32)

    def body(x_ref, w_ref, sx_ref, sw_ref, ids_ref, out_ref,
             comm_f, comm_b, pbuf_f, pbuf_b, wb_ref,
             send_f, recv_f, send_b, recv_b,
             credit_f, credit_b, store_sem_f, store_sem_b):
        r = ids_ref[0]
        left = ids_ref[1]
        right = ids_ref[2]
        scale = sx_ref[0] * sw_ref[0]

        barrier_sem = pltpu.get_barrier_semaphore()
        _sem_signal(barrier_sem, inc=1, device_id=(left,), device_id_type=_MESH)
        _sem_signal(barrier_sem, inc=1, device_id=(right,), device_id_type=_MESH)
        _sem_wait(barrier_sem, 2)

        wb_ref[...] = w_ref[...].astype(jnp.bfloat16)

        def partial_half(c, half):
            xc = x_ref[pl.ds(c * chunk, chunk), :].astype(jnp.bfloat16)
            wh = wb_ref[:, pl.ds(half * nh, nh)]
            return jnp.dot(xc, wh, preferred_element_type=jnp.float32) * scale

        def ring_step(g, do_accum, cf, cb):
            send_slot = lax.rem(g, 2)
            recv_slot = lax.rem(g + 1, 2)

            @pl.when(g >= 1)
            def _():
                _sem_wait(credit_f, 1)
                _sem_wait(credit_b, 1)

            rdma_f = pltpu.make_async_remote_copy(
                src_ref=comm_f.at[send_slot],
                dst_ref=comm_f.at[recv_slot],
                send_sem=send_f.at[send_slot],
                recv_sem=recv_f.at[recv_slot],
                device_id=(right,),
                device_id_type=_MESH,
            )
            rdma_b = pltpu.make_async_remote_copy(
                src_ref=comm_b.at[send_slot],
                dst_ref=comm_b.at[recv_slot],
                send_sem=send_b.at[send_slot],
                recv_sem=recv_b.at[recv_slot],
                device_id=(left,),
                device_id_type=_MESH,
            )
            rdma_f.start()
            rdma_b.start()
            if do_accum:
                pbuf_f[...] = partial_half(cf, 0)
                pbuf_b[...] = partial_half(cb, 1)
            rdma_f.wait()
            rdma_b.wait()
            if do_accum:
                comm_f[recv_slot] = comm_f[recv_slot] + pbuf_f[...]
                comm_b[recv_slot] = comm_b[recv_slot] + pbuf_b[...]
            else:
                st_f = pltpu.make_async_copy(
                    comm_f.at[recv_slot],
                    out_ref.at[pl.ds(cf * chunk, chunk), pl.ds(0, nh)],
                    store_sem_f,
                )
                st_b = pltpu.make_async_copy(
                    comm_b.at[recv_slot],
                    out_ref.at[pl.ds(cb * chunk, chunk), pl.ds(nh, nh)],
                    store_sem_b,
                )
                st_f.start()
                st_b.start()
                st_f.wait()
                st_b.wait()
            _sem_signal(credit_f, inc=1, device_id=(left,),
                        device_id_type=_MESH)
            _sem_signal(credit_b, inc=1, device_id=(right,),
                        device_id_type=_MESH)

        comm_f[0] = partial_half(r, 0)
        comm_b[0] = partial_half(r, 1)

        def rs_step(s, carry):
            cf = lax.rem(r - s - 1 + 2 * N_DEV, N_DEV)
            cb = lax.rem(r + s + 1, N_DEV)
            ring_step(s, True, cf, cb)
            return carry

        lax.fori_loop(0, N_DEV - 1, rs_step, 0)

        red_f = lax.rem(r + 1, N_DEV)
        red_b = lax.rem(r + N_DEV - 1, N_DEV)
        st_f = pltpu.make_async_copy(
            comm_f.at[1],
            out_ref.at[pl.ds(red_f * chunk, chunk), pl.ds(0, nh)],
            store_sem_f,
        )
        st_b = pltpu.make_async_copy(
            comm_b.at[1],
            out_ref.at[pl.ds(red_b * chunk, chunk), pl.ds(nh, nh)],
            store_sem_b,
        )
        st_f.start()
        st_b.start()
        st_f.wait()
        st_b.wait()

        def ag_step(t, carry):
            cf = lax.rem(r - t + 2 * N_DEV, N_DEV)
            cb = lax.rem(r + t, N_DEV)
            ring_step(t + N_DEV - 1, False, cf, cb)
            return carry

        lax.fori_loop(0, N_DEV - 1, ag_step, 0)

        _sem_wait(credit_f, 1)
        _sem_wait(credit_b, 1)

    return pl.pallas_call(
        body,
        out_shape=jax.ShapeDtypeStruct((m, n), jnp.float32),
        in_specs=[
            pl.BlockSpec(memory_space=pltpu.VMEM),
            pl.BlockSpec(memory_space=pltpu.VMEM),
            pl.BlockSpec(memory_space=pltpu.SMEM),
            pl.BlockSpec(memory_space=pltpu.SMEM),
            pl.BlockSpec(memory_space=pltpu.SMEM),
        ],
        out_specs=pl.BlockSpec(memory_space=pl.ANY),
        scratch_shapes=[
            pltpu.VMEM((2, chunk, nh), jnp.float32),
            pltpu.VMEM((2, chunk, nh), jnp.float32),
            pltpu.VMEM((chunk, nh), jnp.float32),
            pltpu.VMEM((chunk, nh), jnp.float32),
            pltpu.VMEM((k_loc, n), jnp.bfloat16),
            pltpu.SemaphoreType.DMA((2,)),
            pltpu.SemaphoreType.DMA((2,)),
            pltpu.SemaphoreType.DMA((2,)),
            pltpu.SemaphoreType.DMA((2,)),
            pltpu.SemaphoreType.REGULAR,
            pltpu.SemaphoreType.REGULAR,
            pltpu.SemaphoreType.DMA,
            pltpu.SemaphoreType.DMA,
        ],
        compiler_params=pltpu.CompilerParams(collective_id=0),
    )(x, w_mat, scale_x, scale_w, ids)


# device time: 1497808 ns/iter; 2.2653x vs baseline; 1.1490x over previous
import jax
import jax.numpy as jnp
import numpy as np
from jax import lax
from jax.experimental import pallas as pl
from jax.experimental.pallas import tpu as pltpu

N_DEV = 32
N_SUB = 4

_sem_signal = getattr(pl, "semaphore_signal", None) or pltpu.semaphore_signal
_sem_wait = getattr(pl, "semaphore_wait", None) or pltpu.semaphore_wait
_DevIdType = getattr(pl, "DeviceIdType", None) or pltpu.DeviceIdType
_MESH = _DevIdType.MESH


def _ring_tables():
    path_yz = []
    for z in range(4):
        ys = range(4) if z % 2 == 0 else range(3, -1, -1)
        path_yz.extend((y, z) for y in ys)
    cycle = [(0, y, z) for (y, z) in path_yz]
    cycle += [(1, y, z) for (y, z) in reversed(path_yz)]
    off = {(0, 0): 0, (1, 0): 1, (1, 1): 2, (0, 1): 3,
           (0, 2): 4, (1, 2): 5, (1, 3): 6, (0, 3): 7}
    perm = np.array([z * 8 + off[(x, y)] for (x, y, z) in cycle],
                    dtype=np.int32)
    inv = np.zeros_like(perm)
    inv[perm] = np.arange(N_DEV, dtype=np.int32)
    return perm, inv


_PERM, _INV = _ring_tables()


def kernel(x, w_mat, scale_x, scale_w):
    m, k_loc = x.shape
    _, n = w_mat.shape
    chunk = m // N_DEV
    nh = n // 2
    ns = n // N_SUB

    my_log = lax.axis_index("i")
    rpos = jnp.take(jnp.asarray(_INV), my_log)
    right_log = jnp.take(jnp.asarray(_PERM), (rpos + 1) % N_DEV)
    left_log = jnp.take(jnp.asarray(_PERM), (rpos + N_DEV - 1) % N_DEV)
    ids = jnp.stack([rpos, left_log, right_log]).astype(jnp.int32)

    def body(x_ref, w_ref, sx_ref, sw_ref, ids_ref, out_ref,
             comm_ref, pbuf_f, pbuf_b, wb_ref,
             send_sems, recv_sems, credit_sems, store_sems):
        r = ids_ref[0]
        left = ids_ref[1]
        right = ids_ref[2]
        scale = sx_ref[0] * sw_ref[0]

        barrier_sem = pltpu.get_barrier_semaphore()
        _sem_signal(barrier_sem, inc=1, device_id=(left,), device_id_type=_MESH)
        _sem_signal(barrier_sem, inc=1, device_id=(right,), device_id_type=_MESH)
        _sem_wait(barrier_sem, 2)

        wb_ref[...] = w_ref[...].astype(jnp.bfloat16)

        def partial_half(c, half):
            xc = x_ref[pl.ds(c * chunk, chunk), :].astype(jnp.bfloat16)
            wh = wb_ref[:, pl.ds(half * nh, nh)]
            return jnp.dot(xc, wh, preferred_element_type=jnp.float32) * scale

        def stream_peers(k):
            return (right, left) if k < 2 else (left, right)

        def pbuf_slice(k):
            pb = pbuf_f if k < 2 else pbuf_b
            return pb[:, pl.ds((k % 2) * ns, ns)]

        def descriptor(k, g, down):
            s0 = lax.rem(g, 2)
            s1 = lax.rem(g + 1, 2)
            return pltpu.make_async_remote_copy(
                src_ref=comm_ref.at[k, s0],
                dst_ref=comm_ref.at[k, s1],
                send_sem=send_sems.at[k, s0],
                recv_sem=recv_sems.at[k, s1],
                device_id=(down,),
                device_id_type=_MESH,
            )

        def stream_step(k, g, is_rs, c_chunk):
            down, up = stream_peers(k)
            recv_slot = lax.rem(g + 1, 2)
            d = descriptor(k, g, down)
            d.wait_recv()
            if is_rs:
                comm_ref[k, recv_slot] = (
                    comm_ref[k, recv_slot] + pbuf_slice(k)
                )
            else:
                st = pltpu.make_async_copy(
                    comm_ref.at[k, recv_slot],
                    out_ref.at[pl.ds(c_chunk * chunk, chunk),
                               pl.ds(k * ns, ns)],
                    store_sems.at[k],
                )
                st.start()
                st.wait()
            d.wait_send()
            _sem_signal(credit_sems.at[k], inc=1, device_id=(up,),
                        device_id_type=_MESH)

            @pl.when(g < 2 * (N_DEV - 1) - 1)
            def _():
                _sem_wait(credit_sems.at[k], 1)
                descriptor(k, g + 1, down).start()

        pbuf_f[...] = partial_half(r, 0)
        pbuf_b[...] = partial_half(r, 1)
        for k in range(N_SUB):
            comm_ref[k, 0] = pbuf_slice(k)
        for k in range(N_SUB):
            descriptor(k, 0, stream_peers(k)[0]).start()

        def rs_step(s, carry):
            cf = lax.rem(r - s - 1 + 2 * N_DEV, N_DEV)
            cb = lax.rem(r + s + 1, N_DEV)
            pbuf_f[...] = partial_half(cf, 0)
            pbuf_b[...] = partial_half(cb, 1)
            stream_step(0, s, True, 0)
            stream_step(2, s, True, 0)
            stream_step(1, s, True, 0)
            stream_step(3, s, True, 0)
            return carry

        lax.fori_loop(0, N_DEV - 1, rs_step, 0)

        red_f = lax.rem(r + 1, N_DEV)
        red_b = lax.rem(r + N_DEV - 1, N_DEV)
        for k in range(N_SUB):
            red = red_f if k < 2 else red_b
            st = pltpu.make_async_copy(
                comm_ref.at[k, 1],
                out_ref.at[pl.ds(red * chunk, chunk), pl.ds(k * ns, ns)],
                store_sems.at[k],
            )
            st.start()
            st.wait()

        def ag_step(t, carry):
            g = t + N_DEV - 1
            cf = lax.rem(r - t + 2 * N_DEV, N_DEV)
            cb = lax.rem(r + t, N_DEV)
            stream_step(0, g, False, cf)
            stream_step(2, g, False, cb)
            stream_step(1, g, False, cf)
            stream_step(3, g, False, cb)
            return carry

        lax.fori_loop(0, N_DEV - 1, ag_step, 0)

        for k in range(N_SUB):
            _sem_wait(credit_sems.at[k], 1)

    return pl.pallas_call(
        body,
        out_shape=jax.ShapeDtypeStruct((m, n), jnp.float32),
        in_specs=[
            pl.BlockSpec(memory_space=pltpu.VMEM),
            pl.BlockSpec(memory_space=pltpu.VMEM),
            pl.BlockSpec(memory_space=pltpu.SMEM),
            pl.BlockSpec(memory_space=pltpu.SMEM),
            pl.BlockSpec(memory_space=pltpu.SMEM),
        ],
        out_specs=pl.BlockSpec(memory_space=pl.ANY),
        scratch_shapes=[
            pltpu.VMEM((N_SUB, 2, chunk, ns), jnp.float32),
            pltpu.VMEM((chunk, nh), jnp.float32),
            pltpu.VMEM((chunk, nh), jnp.float32),
            pltpu.VMEM((k_loc, n), jnp.bfloat16),
            pltpu.SemaphoreType.DMA((N_SUB, 2)),
            pltpu.SemaphoreType.DMA((N_SUB, 2)),
            pltpu.SemaphoreType.REGULAR((N_SUB,)),
            pltpu.SemaphoreType.DMA((N_SUB,)),
        ],
        compiler_params=pltpu.CompilerParams(collective_id=0),
    )(x, w_mat, scale_x, scale_w, ids)


# device time: 1071115 ns/iter; 3.1676x vs baseline; 1.3984x over previous
import jax
import jax.numpy as jnp
import numpy as np
from jax import lax
from jax.experimental import pallas as pl
from jax.experimental.pallas import tpu as pltpu

N_DEV = 32
N_STREAMS = 6

_sem_signal = getattr(pl, "semaphore_signal", None) or pltpu.semaphore_signal
_sem_wait = getattr(pl, "semaphore_wait", None) or pltpu.semaphore_wait
_DevIdType = getattr(pl, "DeviceIdType", None) or pltpu.DeviceIdType
_MESH = _DevIdType.MESH

_CYCLES = np.array([
    [0, 3, 4, 7, 15, 12, 11, 8, 16, 19, 20, 23, 31, 28, 27, 24,
     25, 26, 29, 30, 22, 21, 18, 17, 9, 10, 13, 14, 6, 5, 2, 1],
    [0, 8, 16, 24, 25, 17, 18, 26, 27, 28, 31, 30, 29, 21, 13, 14,
     22, 23, 15, 7, 6, 5, 4, 12, 20, 19, 11, 10, 9, 1, 2, 3],
    [0, 8, 11, 12, 20, 28, 29, 21, 22, 30, 31, 23, 15, 14, 6, 7,
     4, 3, 2, 5, 13, 10, 18, 19, 16, 24, 27, 26, 25, 17, 9, 1],
], dtype=np.int32)
_INVS = np.zeros_like(_CYCLES)
for _c in range(3):
    _INVS[_c, _CYCLES[_c]] = np.arange(N_DEV, dtype=np.int32)

_WIDTH_BLOCKS = [11, 11, 11, 10, 11, 10]
_BLK = 128
_WIDTHS = [w * _BLK for w in _WIDTH_BLOCKS]
_OFFS = np.concatenate([[0], np.cumsum(_WIDTHS)]).tolist()


def kernel(x, w_mat, scale_x, scale_w):
    m, k_loc = x.shape
    _, n = w_mat.shape
    chunk = m // N_DEV
    assert sum(_WIDTHS) == n

    my_log = lax.axis_index("i")
    id_list = []
    for c in range(3):
        rc = jnp.take(jnp.asarray(_INVS[c]), my_log)
        right = jnp.take(jnp.asarray(_CYCLES[c]), (rc + 1) % N_DEV)
        left = jnp.take(jnp.asarray(_CYCLES[c]), (rc + N_DEV - 1) % N_DEV)
        id_list += [rc, left, right]
    ids = jnp.stack(id_list).astype(jnp.int32)

    def body(x_ref, w_ref, sx_ref, sw_ref, ids_ref, out_ref, *scr):
        comm = scr[0:6]
        pbuf = scr[6:12]
        wb_ref = scr[12]
        send_sems = scr[13]
        recv_sems = scr[14]
        credit_sems = scr[15]
        store_sems = scr[16]

        scale = sx_ref[0] * sw_ref[0]

        def cyc_ids(c):
            return ids_ref[3 * c], ids_ref[3 * c + 1], ids_ref[3 * c + 2]

        def stream_peers(k):
            _, left, right = cyc_ids(k // 2)
            return (right, left) if k % 2 == 0 else (left, right)

        barrier_sem = pltpu.get_barrier_semaphore()
        for c in range(3):
            _, left, right = cyc_ids(c)
            _sem_signal(barrier_sem, inc=1, device_id=(left,),
                        device_id_type=_MESH)
            _sem_signal(barrier_sem, inc=1, device_id=(right,),
                        device_id_type=_MESH)
        _sem_wait(barrier_sem, 6)

        wb_ref[...] = w_ref[...].astype(jnp.bfloat16)

        def fill_partial(k, c_chunk):
            xc = x_ref[pl.ds(c_chunk * chunk, chunk), :].astype(jnp.bfloat16)
            wh = wb_ref[:, pl.ds(_OFFS[k], _WIDTHS[k])]
            pbuf[k][...] = jnp.dot(
                xc, wh, preferred_element_type=jnp.float32) * scale

        def rs_chunk(k, s):
            r = cyc_ids(k // 2)[0]
            if k % 2 == 0:
                return lax.rem(r - s - 1 + 2 * N_DEV, N_DEV)
            return lax.rem(r + s + 1, N_DEV)

        def ag_chunk(k, t):
            r = cyc_ids(k // 2)[0]
            if k % 2 == 0:
                return lax.rem(r - t + 2 * N_DEV, N_DEV)
            return lax.rem(r + t, N_DEV)

        def descriptor(k, g):
            down = stream_peers(k)[0]
            s0 = lax.rem(g, 2)
            s1 = lax.rem(g + 1, 2)
            return pltpu.make_async_remote_copy(
                src_ref=comm[k].at[s0],
                dst_ref=comm[k].at[s1],
                send_sem=send_sems.at[k, s0],
                recv_sem=recv_sems.at[k, s1],
                device_id=(down,),
                device_id_type=_MESH,
            )

        def stream_step(k, g, is_rs, c_store):
            up = stream_peers(k)[1]
            recv_slot = lax.rem(g + 1, 2)
            d = descriptor(k, g)
            d.wait_recv()
            if is_rs:
                comm[k][recv_slot] = comm[k][recv_slot] + pbuf[k][...]
            else:
                st = pltpu.make_async_copy(
                    comm[k].at[recv_slot],
                    out_ref.at[pl.ds(c_store * chunk, chunk),
                               pl.ds(_OFFS[k], _WIDTHS[k])],
                    store_sems.at[k],
                )
                st.start()
                st.wait()
            d.wait_send()
            _sem_signal(credit_sems.at[k], inc=1, device_id=(up,),
                        device_id_type=_MESH)

            @pl.when(g < 2 * (N_DEV - 1) - 1)
            def _():
                _sem_wait(credit_sems.at[k], 1)
                descriptor(k, g + 1).start()

        for k in range(N_STREAMS):
            fill_partial(k, cyc_ids(k // 2)[0])
            comm[k][0] = pbuf[k][...]
        for k in range(N_STREAMS):
            descriptor(k, 0).start()

        def rs_step(s, carry):
            for k in range(N_STREAMS):
                fill_partial(k, rs_chunk(k, s))
            for k in range(N_STREAMS):
                stream_step(k, s, True, 0)
            return carry

        lax.fori_loop(0, N_DEV - 1, rs_step, 0)

        for k in range(N_STREAMS):
            r = cyc_ids(k // 2)[0]
            red = lax.rem(r + 1, N_DEV) if k % 2 == 0 \
                else lax.rem(r + N_DEV - 1, N_DEV)
            st = pltpu.make_async_copy(
                comm[k].at[1],
                out_ref.at[pl.ds(red * chunk, chunk),
                           pl.ds(_OFFS[k], _WIDTHS[k])],
                store_sems.at[k],
            )
            st.start()
            st.wait()

        def ag_step(t, carry):
            g = t + N_DEV - 1
            for k in range(N_STREAMS):
                stream_step(k, g, False, ag_chunk(k, t))
            return carry

        lax.fori_loop(0, N_DEV - 1, ag_step, 0)

        for k in range(N_STREAMS):
            _sem_wait(credit_sems.at[k], 1)

    scratch = (
        [pltpu.VMEM((2, chunk, w), jnp.float32) for w in _WIDTHS]
        + [pltpu.VMEM((chunk, w), jnp.float32) for w in _WIDTHS]
        + [
            pltpu.VMEM((k_loc, n), jnp.bfloat16),
            pltpu.SemaphoreType.DMA((N_STREAMS, 2)),
            pltpu.SemaphoreType.DMA((N_STREAMS, 2)),
            pltpu.SemaphoreType.REGULAR((N_STREAMS,)),
            pltpu.SemaphoreType.DMA((N_STREAMS,)),
        ]
    )

    return pl.pallas_call(
        body,
        out_shape=jax.ShapeDtypeStruct((m, n), jnp.float32),
        in_specs=[
            pl.BlockSpec(memory_space=pltpu.VMEM),
            pl.BlockSpec(memory_space=pltpu.VMEM),
            pl.BlockSpec(memory_space=pltpu.SMEM),
            pl.BlockSpec(memory_space=pltpu.SMEM),
            pl.BlockSpec(memory_space=pltpu.SMEM),
        ],
        out_specs=pl.BlockSpec(memory_space=pl.ANY),
        scratch_shapes=scratch,
        compiler_params=pltpu.CompilerParams(collective_id=0),
    )(x, w_mat, scale_x, scale_w, ids)


# device time: 1068605 ns/iter; 3.1751x vs baseline; 1.0023x over previous
import jax
import jax.numpy as jnp
import numpy as np
from jax import lax
from jax.experimental import pallas as pl
from jax.experimental.pallas import tpu as pltpu

N_DEV = 32
N_STREAMS = 6

_sem_signal = getattr(pl, "semaphore_signal", None) or pltpu.semaphore_signal
_sem_wait = getattr(pl, "semaphore_wait", None) or pltpu.semaphore_wait
_DevIdType = getattr(pl, "DeviceIdType", None) or pltpu.DeviceIdType
_MESH = _DevIdType.MESH

_CYCLES = np.array([
    [0, 3, 4, 7, 15, 12, 11, 8, 16, 19, 20, 23, 31, 28, 27, 24,
     25, 26, 29, 30, 22, 21, 18, 17, 9, 10, 13, 14, 6, 5, 2, 1],
    [0, 8, 16, 24, 25, 17, 18, 26, 27, 28, 31, 30, 29, 21, 13, 14,
     22, 23, 15, 7, 6, 5, 4, 12, 20, 19, 11, 10, 9, 1, 2, 3],
    [0, 8, 11, 12, 20, 28, 29, 21, 22, 30, 31, 23, 15, 14, 6, 7,
     4, 3, 2, 5, 13, 10, 18, 19, 16, 24, 27, 26, 25, 17, 9, 1],
], dtype=np.int32)
_INVS = np.zeros_like(_CYCLES)
for _c in range(3):
    _INVS[_c, _CYCLES[_c]] = np.arange(N_DEV, dtype=np.int32)

_WIDTH_BLOCKS = [11, 11, 11, 10, 11, 10]
_BLK = 128
_WIDTHS = [w * _BLK for w in _WIDTH_BLOCKS]
_OFFS = np.concatenate([[0], np.cumsum(_WIDTHS)]).tolist()


def kernel(x, w_mat, scale_x, scale_w):
    m, k_loc = x.shape
    _, n = w_mat.shape
    chunk = m // N_DEV
    assert sum(_WIDTHS) == n

    my_log = lax.axis_index("i")
    id_list = []
    for c in range(3):
        rc = jnp.take(jnp.asarray(_INVS[c]), my_log)
        right = jnp.take(jnp.asarray(_CYCLES[c]), (rc + 1) % N_DEV)
        left = jnp.take(jnp.asarray(_CYCLES[c]), (rc + N_DEV - 1) % N_DEV)
        id_list += [rc, left, right]
    ids = jnp.stack(id_list).astype(jnp.int32)

    def body(x_ref, w_ref, sx_ref, sw_ref, ids_ref, out_ref, *scr):
        comm = scr[0:6]
        pbuf = scr[6:12]
        wb_ref = scr[12]
        send_sems = scr[13]
        recv_sems = scr[14]
        credit_sems = scr[15]
        store_sems = scr[16]

        scale = sx_ref[0] * sw_ref[0]

        def cyc_ids(c):
            return ids_ref[3 * c], ids_ref[3 * c + 1], ids_ref[3 * c + 2]

        def stream_peers(k):
            _, left, right = cyc_ids(k // 2)
            return (right, left) if k % 2 == 0 else (left, right)

        barrier_sem = pltpu.get_barrier_semaphore()
        for c in range(3):
            _, left, right = cyc_ids(c)
            _sem_signal(barrier_sem, inc=1, device_id=(left,),
                        device_id_type=_MESH)
            _sem_signal(barrier_sem, inc=1, device_id=(right,),
                        device_id_type=_MESH)
        _sem_wait(barrier_sem, 6)

        wb_ref[...] = w_ref[...].astype(jnp.bfloat16)

        def fill_partial(k, c_chunk):
            xc = x_ref[pl.ds(c_chunk * chunk, chunk), :].astype(jnp.bfloat16)
            wh = wb_ref[:, pl.ds(_OFFS[k], _WIDTHS[k])]
            pbuf[k][...] = jnp.dot(
                xc, wh, preferred_element_type=jnp.float32) * scale

        def rs_chunk(k, s):
            r = cyc_ids(k // 2)[0]
            if k % 2 == 0:
                return lax.rem(r - s - 1 + 2 * N_DEV, N_DEV)
            return lax.rem(r + s + 1, N_DEV)

        def ag_chunk(k, t):
            r = cyc_ids(k // 2)[0]
            if k % 2 == 0:
                return lax.rem(r - t + 2 * N_DEV, N_DEV)
            return lax.rem(r + t, N_DEV)

        def descriptor(k, g):
            down = stream_peers(k)[0]
            s0 = lax.rem(g, 2)
            s1 = lax.rem(g + 1, 2)
            return pltpu.make_async_remote_copy(
                src_ref=comm[k].at[s0],
                dst_ref=comm[k].at[s1],
                send_sem=send_sems.at[k, s0],
                recv_sem=recv_sems.at[k, s1],
                device_id=(down,),
                device_id_type=_MESH,
            )

        def store_desc(k, slot, c_store):
            return pltpu.make_async_copy(
                comm[k].at[slot],
                out_ref.at[pl.ds(c_store * chunk, chunk),
                           pl.ds(_OFFS[k], _WIDTHS[k])],
                store_sems.at[k],
            )

        def stream_step(k, g, is_rs, c_store):
            up = stream_peers(k)[1]
            recv_slot = lax.rem(g + 1, 2)
            d = descriptor(k, g)
            d.wait_recv()
            if is_rs:
                comm[k][recv_slot] = comm[k][recv_slot] + pbuf[k][...]
            else:
                st = store_desc(k, recv_slot, c_store)

                @pl.when(g >= N_DEV)
                def _():
                    st.wait()
                st.start()
            d.wait_send()
            _sem_signal(credit_sems.at[k], inc=1, device_id=(up,),
                        device_id_type=_MESH)

            @pl.when(g < 2 * (N_DEV - 1) - 1)
            def _():
                _sem_wait(credit_sems.at[k], 1)
                descriptor(k, g + 1).start()

        for k in range(N_STREAMS):
            fill_partial(k, cyc_ids(k // 2)[0])
            comm[k][0] = pbuf[k][...]
        for k in range(N_STREAMS):
            descriptor(k, 0).start()

        def rs_step(s, carry):
            for k in range(N_STREAMS):
                fill_partial(k, rs_chunk(k, s))
            for k in range(N_STREAMS):
                stream_step(k, s, True, 0)
            return carry

        lax.fori_loop(0, N_DEV - 1, rs_step, 0)

        for k in range(N_STREAMS):
            r = cyc_ids(k // 2)[0]
            red = lax.rem(r + 1, N_DEV) if k % 2 == 0 \
                else lax.rem(r + N_DEV - 1, N_DEV)
            st = pltpu.make_async_copy(
                comm[k].at[1],
                out_ref.at[pl.ds(red * chunk, chunk),
                           pl.ds(_OFFS[k], _WIDTHS[k])],
                store_sems.at[k],
            )
            st.start()
            st.wait()

        def ag_step(t, carry):
            g = t + N_DEV - 1
            for k in range(N_STREAMS):
                stream_step(k, g, False, ag_chunk(k, t))
            return carry

        lax.fori_loop(0, N_DEV - 1, ag_step, 0)

        for k in range(N_STREAMS):
            store_desc(k, 0, 0).wait()
            _sem_wait(credit_sems.at[k], 1)

    scratch = (
        [pltpu.VMEM((2, chunk, w), jnp.float32) for w in _WIDTHS]
        + [pltpu.VMEM((chunk, w), jnp.float32) for w in _WIDTHS]
        + [
            pltpu.VMEM((k_loc, n), jnp.bfloat16),
            pltpu.SemaphoreType.DMA((N_STREAMS, 2)),
            pltpu.SemaphoreType.DMA((N_STREAMS, 2)),
            pltpu.SemaphoreType.REGULAR((N_STREAMS,)),
            pltpu.SemaphoreType.DMA((N_STREAMS,)),
        ]
    )

    return pl.pallas_call(
        body,
        out_shape=jax.ShapeDtypeStruct((m, n), jnp.float32),
        in_specs=[
            pl.BlockSpec(memory_space=pltpu.VMEM),
            pl.BlockSpec(memory_space=pltpu.VMEM),
            pl.BlockSpec(memory_space=pltpu.SMEM),
            pl.BlockSpec(memory_space=pltpu.SMEM),
            pl.BlockSpec(memory_space=pltpu.SMEM),
        ],
        out_specs=pl.BlockSpec(memory_space=pl.ANY),
        scratch_shapes=scratch,
        compiler_params=pltpu.CompilerParams(collective_id=0),
    )(x, w_mat, scale_x, scale_w, ids)
